# baseline (speedup 1.0000x reference)
import functools

import jax
import jax.numpy as jnp
from jax.experimental import pallas as pl

_EPS = 1e-06
_CAPACITY_FACTOR = 1.0


def _gate_kernel(x_ref, w_ref, b_ref, o_ref, *, capacity):
    bsz, cs, dim = x_ref.shape
    ne = w_ref.shape[1]
    x2 = x_ref[...].reshape(bsz * cs, dim)
    logits = jnp.dot(x2, w_ref[...], preferred_element_type=jnp.float32)
    logits = logits + b_ref[...]
    m = jnp.max(logits, axis=-1, keepdims=True)
    e = jnp.exp(logits - m)
    g = e / jnp.sum(e, axis=-1, keepdims=True)
    g3 = g.reshape(bsz, cs, ne)
    den = jnp.sum(g3, axis=0, keepdims=True) + _EPS
    o_ref[...] = g3 * (capacity / den)


@functools.partial(jax.jit, static_argnames=())
def kernel(X, W, b):
    bsz, seq, dim = X.shape
    ne = W.shape[1]
    capacity = float(int(_CAPACITY_FACTOR * bsz))
    cs = 512
    b2 = b.reshape(1, ne)
    grid = (seq // cs,)
    return pl.pallas_call(
        functools.partial(_gate_kernel, capacity=capacity),
        grid=grid,
        in_specs=[
            pl.BlockSpec((bsz, cs, dim), lambda i: (0, i, 0)),
            pl.BlockSpec((dim, ne), lambda i: (0, 0)),
            pl.BlockSpec((1, ne), lambda i: (0, 0)),
        ],
        out_specs=pl.BlockSpec((bsz, cs, ne), lambda i: (0, i, 0)),
        out_shape=jax.ShapeDtypeStruct((bsz, seq, ne), jnp.float32),
    )(X, W, b2)


# speedup vs baseline: 1.0172x; 1.0172x over previous
import functools

import jax
import jax.numpy as jnp
from jax.experimental import pallas as pl

_EPS = 1e-06
_CAPACITY_FACTOR = 1.0


def _gate_kernel(x_ref, w_ref, b_ref, o_ref, *, capacity):
    bsz, cs, dim = x_ref.shape
    ne = w_ref.shape[1]
    x2 = x_ref[...].reshape(bsz * cs, dim)
    logits = jnp.dot(x2, w_ref[...], preferred_element_type=jnp.float32)
    logits = logits + b_ref[...]
    m = jnp.max(logits, axis=-1, keepdims=True)
    e = jnp.exp(logits - m)
    g = e / jnp.sum(e, axis=-1, keepdims=True)
    g3 = g.reshape(bsz, cs, ne)
    den = jnp.sum(g3, axis=0, keepdims=True) + _EPS
    o_ref[...] = g3 * (capacity / den)


@functools.partial(jax.jit, static_argnames=())
def kernel(X, W, b):
    bsz, seq, dim = X.shape
    ne = W.shape[1]
    capacity = float(int(_CAPACITY_FACTOR * bsz))
    cs = 256
    b2 = b.reshape(1, ne)
    grid = (seq // cs,)
    return pl.pallas_call(
        functools.partial(_gate_kernel, capacity=capacity),
        grid=grid,
        in_specs=[
            pl.BlockSpec((bsz, cs, dim), lambda i: (0, i, 0)),
            pl.BlockSpec((dim, ne), lambda i: (0, 0)),
            pl.BlockSpec((1, ne), lambda i: (0, 0)),
        ],
        out_specs=pl.BlockSpec((bsz, cs, ne), lambda i: (0, i, 0)),
        out_shape=jax.ShapeDtypeStruct((bsz, seq, ne), jnp.float32),
    )(X, W, b2)


# speedup vs baseline: 1.0198x; 1.0027x over previous
import functools

import jax
import jax.numpy as jnp
from jax.experimental import pallas as pl

_EPS = 1e-06
_CAPACITY_FACTOR = 1.0


def _gate_kernel(x0_ref, x1_ref, x2_ref, x3_ref, w_ref, b_ref, o_ref, *, capacity):
    bsz, cs, dpart = x0_ref.shape
    ne = w_ref.shape[1]
    logits = b_ref[...]
    for j, xr in enumerate((x0_ref, x1_ref, x2_ref, x3_ref)):
        xp = xr[...].reshape(bsz * cs, dpart)
        wp = w_ref[j * dpart:(j + 1) * dpart, :]
        logits = logits + jnp.dot(xp, wp, preferred_element_type=jnp.float32)
    m = jnp.max(logits, axis=-1, keepdims=True)
    e = jnp.exp(logits - m)
    g = e / jnp.sum(e, axis=-1, keepdims=True)
    g3 = g.reshape(bsz, cs, ne)
    den = jnp.sum(g3, axis=0, keepdims=True) + _EPS
    o_ref[...] = g3 * (capacity / den)


@functools.partial(jax.jit, static_argnames=())
def kernel(X, W, b):
    bsz, seq, dim = X.shape
    ne = W.shape[1]
    capacity = float(int(_CAPACITY_FACTOR * bsz))
    cs = 256
    b2 = b.reshape(1, ne)
    grid = (seq // cs,)
    return pl.pallas_call(
        functools.partial(_gate_kernel, capacity=capacity),
        grid=grid,
        in_specs=[
            pl.BlockSpec((bsz, cs, dim // 4), lambda i, j=j: (0, i, j))
            for j in range(4)
        ] + [
            pl.BlockSpec((dim, ne), lambda i: (0, 0)),
            pl.BlockSpec((1, ne), lambda i: (0, 0)),
        ],
        out_specs=pl.BlockSpec((bsz, cs, ne), lambda i: (0, i, 0)),
        out_shape=jax.ShapeDtypeStruct((bsz, seq, ne), jnp.float32),
    )(X, X, X, X, W, b2)
